# baseline (speedup 1.0000x reference)
import functools
import math

import jax
import jax.numpy as jnp
import numpy as np
from jax import lax
from jax.experimental import pallas as pl
from jax.experimental.pallas import tpu as pltpu

_ROTS = ((13, 15, 26, 6), (17, 29, 16, 24))
_TINY = np.float32(np.finfo(np.float32).tiny)
_INTMAX = np.int32(np.iinfo(np.int32).max)


def _threefry_bits(p):
    k0 = jnp.uint32(0)
    k1 = jnp.uint32(1)
    ks = (k0, k1, jnp.uint32(0x1BD11BDA) ^ k0 ^ k1)
    x0 = jnp.full_like(p, k0)
    x1 = p + k1
    for i in range(5):
        for r in _ROTS[i % 2]:
            x0 = x0 + x1
            x1 = (x1 << jnp.uint32(r)) | (x1 >> jnp.uint32(32 - r))
            x1 = x0 ^ x1
        x0 = x0 + ks[(i + 1) % 3]
        x1 = x1 + ks[(i + 2) % 3] + jnp.uint32(i + 1)
    return x0 ^ x1


def _gumbel(bits):
    fb = (bits >> jnp.uint32(9)) | jnp.uint32(0x3F800000)
    f = lax.bitcast_convert_type(fb, jnp.float32) - jnp.float32(1.0)
    u = jnp.maximum(f + _TINY, _TINY)
    return -jnp.log(-jnp.log(u))


def _pass1_kernel(x_ref, max_out, idx_out, mrun, irun, *, blk, ncb, vocab):
    i = pl.program_id(0)

    @pl.when(i == 0)
    def _init():
        mrun[...] = jnp.full_like(mrun, -jnp.inf)
        irun[...] = jnp.zeros_like(irun)

    x = x_ref[...]
    col = lax.broadcasted_iota(jnp.int32, x.shape, 1) + i * blk
    xm = jnp.where(col < vocab, x, -jnp.inf)
    bm = jnp.max(xm, axis=1, keepdims=True)
    bi = jnp.min(jnp.where(xm == bm, col, _INTMAX), axis=1, keepdims=True)
    better = bm > mrun[...]
    irun[...] = jnp.where(better, bi, irun[...])
    mrun[...] = jnp.where(better, bm, mrun[...])

    @pl.when(i == ncb - 1)
    def _fin():
        max_out[...] = mrun[...]
        idx_out[...] = irun[...]


def _pass2_kernel(t_ref, rowmax_ref, greedy_ref, x_ref, out_ref, yrun, irun,
                  *, blk, ncb, vocab):
    i = pl.program_id(0)

    @pl.when(i == 0)
    def _init():
        yrun[...] = jnp.full_like(yrun, -jnp.inf)
        irun[...] = jnp.zeros_like(irun)

    x = x_ref[...]
    col = lax.broadcasted_iota(jnp.int32, x.shape, 1) + i * blk
    row = lax.broadcasted_iota(jnp.int32, x.shape, 0)
    p = (row * vocab + col).astype(jnp.uint32)
    g = _gumbel(_threefry_bits(p))

    safe_t = jnp.maximum(t_ref[...], jnp.float32(1e-6))
    shift = rowmax_ref[...] / safe_t
    y = (x / safe_t - shift) + g
    y = jnp.where(col < vocab, y, -jnp.inf)
    bm = jnp.max(y, axis=1, keepdims=True)
    bi = jnp.min(jnp.where(y == bm, col, _INTMAX), axis=1, keepdims=True)
    better = bm > yrun[...]
    irun[...] = jnp.where(better, bi, irun[...])
    yrun[...] = jnp.where(better, bm, yrun[...])

    @pl.when(i == ncb - 1)
    def _fin():
        out_ref[...] = jnp.where(t_ref[...] <= jnp.float32(1e-6),
                                 greedy_ref[...], irun[...])


@functools.partial(jax.jit, static_argnames=("blk",))
def _sample(logits, temperatures, blk=16384):
    rows, vocab = logits.shape
    ncb = math.ceil(vocab / blk)
    t2 = temperatures.reshape(rows, 1)

    rowmax, greedy = pl.pallas_call(
        functools.partial(_pass1_kernel, blk=blk, ncb=ncb, vocab=vocab),
        grid=(ncb,),
        in_specs=[pl.BlockSpec((rows, blk), lambda i: (0, i))],
        out_specs=[pl.BlockSpec((rows, 1), lambda i: (0, 0)),
                   pl.BlockSpec((rows, 1), lambda i: (0, 0))],
        out_shape=[jax.ShapeDtypeStruct((rows, 1), jnp.float32),
                   jax.ShapeDtypeStruct((rows, 1), jnp.int32)],
        scratch_shapes=[pltpu.VMEM((rows, 1), jnp.float32),
                        pltpu.VMEM((rows, 1), jnp.int32)],
    )(logits)

    out = pl.pallas_call(
        functools.partial(_pass2_kernel, blk=blk, ncb=ncb, vocab=vocab),
        grid=(ncb,),
        in_specs=[pl.BlockSpec((rows, 1), lambda i: (0, 0)),
                  pl.BlockSpec((rows, 1), lambda i: (0, 0)),
                  pl.BlockSpec((rows, 1), lambda i: (0, 0)),
                  pl.BlockSpec((rows, blk), lambda i: (0, i))],
        out_specs=pl.BlockSpec((rows, 1), lambda i: (0, 0)),
        out_shape=jax.ShapeDtypeStruct((rows, 1), jnp.int32),
        scratch_shapes=[pltpu.VMEM((rows, 1), jnp.float32),
                        pltpu.VMEM((rows, 1), jnp.int32)],
    )(t2, rowmax, greedy, logits)

    return out.reshape(rows)


def kernel(logits, temperatures):
    if logits.ndim == 1:
        logits = logits[None, :]
    temperatures = jnp.reshape(temperatures, (-1,))
    if temperatures.shape[0] == 1 and logits.shape[0] > 1:
        temperatures = jnp.repeat(temperatures, logits.shape[0])
    return _sample(logits, temperatures)
